# baseline (speedup 1.0000x reference)
import functools

import jax
import jax.numpy as jnp
from jax import lax
from jax.experimental import pallas as pl
from jax.experimental.pallas import tpu as pltpu
from jax.experimental.pallas import tpu_sc as plsc

N, E, D, DE, HID = 10000, 320000, 128, 16, 256

NC, NS = 2, 16
DH = D // NC
EPW = E // NS
CHUNK = 80
NCHUNK = EPW // CHUNK
NPAD = 10240
ROWS_PER_SUB = NPAD // NS


def _edge_lin_body(ea_ref, w_ref, b_ref, out0_ref, out1_ref, out2_ref,
                   out3_ref):
    ea = ea_ref[...]
    outs = (out0_ref, out1_ref, out2_ref, out3_ref)
    for c in range(NC):
        r = (jnp.dot(ea, w_ref[c], preferred_element_type=jnp.float32)
             + b_ref[c])
        for m in range(4):
            outs[m][c] = r[:, m * D:(m + 1) * D]


_BE8 = 1000


def _edge_lin(edge_attr8, lw, lb):
    eye8 = jnp.eye(8, dtype=jnp.float32)
    wp = jnp.stack([
        jnp.kron(eye8, lw[:, :DH]),
        jnp.kron(eye8, lw[:, DH:]),
    ])
    bp = jnp.stack([
        jnp.tile(lb[:DH], 8).reshape(1, 8 * DH),
        jnp.tile(lb[DH:], 8).reshape(1, 8 * DH),
    ])
    out = jax.ShapeDtypeStruct((NC, E // 8, D), jnp.float32)
    return pl.pallas_call(
        _edge_lin_body,
        grid=(E // 8 // _BE8,),
        in_specs=[
            pl.BlockSpec((_BE8, 8 * DE), lambda i: (i, 0)),
            pl.BlockSpec((NC, 8 * DE, 8 * DH), lambda i: (0, 0, 0)),
            pl.BlockSpec((NC, 1, 8 * DH), lambda i: (0, 0, 0)),
        ],
        out_specs=[pl.BlockSpec((NC, _BE8, D), lambda i: (0, i, 0))] * 4,
        out_shape=[out] * 4,
    )(edge_attr8, wp, bp)


NSLOT = 3


def _sc_agg_body(h_hbm, e0_hbm, e1_hbm, e2_hbm, e3_hbm, src_hbm, dst_hbm,
                 out_hbm,
                 src_v, dst_v,
                 idx0, idx1, idx2,
                 hbuf0, hbuf1, hbuf2,
                 ebuf0, ebuf1, ebuf2,
                 mbuf0, mbuf1, mbuf2,
                 agg,
                 sem_h0, sem_h1, sem_h2,
                 sem_e0, sem_e1, sem_e2,
                 sem_m0, sem_m1, sem_m2):
    e_hbms = (e0_hbm, e1_hbm, e2_hbm, e3_hbm)
    idxs = (idx0, idx1, idx2)
    hbufs = (hbuf0, hbuf1, hbuf2)
    ebufs = (ebuf0, ebuf1, ebuf2)
    mbufs = (mbuf0, mbuf1, mbuf2)
    sem_hs = (sem_h0, sem_h1, sem_h2)
    sem_es = (sem_e0, sem_e1, sem_e2)
    sem_ms = (sem_m0, sem_m1, sem_m2)
    c = lax.axis_index("c")
    s = lax.axis_index("s")

    pltpu.sync_copy(src_hbm.at[s], src_v)
    pltpu.sync_copy(dst_hbm.at[s], dst_v)

    zero16 = jnp.zeros((16,), jnp.float32)

    def zrow(i, carry):
        for j in range(DH // 16):
            mbuf0[i, pl.ds(j * 16, 16)] = zero16
        return carry

    lax.fori_loop(0, CHUNK, zrow, 0)
    base_n = s * ROWS_PER_SUB
    for r in range(ROWS_PER_SUB // CHUNK):
        pltpu.sync_copy(mbuf0, agg.at[pl.ds(base_n + r * CHUNK, CHUNK), :])
    plsc.subcore_barrier()

    edge_base = s * EPW
    bufs = tuple(
        (idxs[k], hbufs[k], ebufs[k], mbufs[k], sem_hs[k], sem_es[k],
         sem_ms[k])
        for k in range(NSLOT))

    def start(g, slot):
        idx, hb, eb, _, sh, se, _sm = bufs[slot]

        def irow(k, carry):
            dsl = pl.ds(k * 16, 16)
            idx[dsl] = src_v[g, dsl] * 2 + c
            return carry

        lax.fori_loop(0, CHUNK // 16, irow, 0)
        pltpu.async_copy(h_hbm.at[idx], hb, sh)
        off8 = (edge_base + g * CHUNK) // 8
        for m in range(4):
            pltpu.async_copy(
                e_hbms[m].at[c, pl.ds(off8, CHUNK // 8), :], eb.at[m], se)

    def finish(g, slot):
        idx, hb, eb, mb, sh, se, sm = bufs[slot]
        pltpu.make_async_copy(h_hbm.at[idx], hb, sh).wait()
        for m in range(4):
            pltpu.make_async_copy(
                e_hbms[m].at[c, pl.ds(0, CHUNK // 8), :], eb.at[m],
                se).wait()

        @pl.when(jnp.asarray(g >= NSLOT))
        def _():
            pltpu.make_async_copy(mb, agg.at[dst_v.at[g]], sm).wait()

        @plsc.parallel_loop(0, CHUNK // 8, unroll=2)
        def mrow(j):
            for m in range(4):
                l = 8 * j + 2 * m
                for k in range(DH // 16):
                    dsl = pl.ds(k * 16, 16)
                    mb[l, dsl] = jnp.maximum(
                        hb[l, dsl] + eb[m, j, dsl], 0.0)
                    mb[l + 1, dsl] = jnp.maximum(
                        hb[l + 1, dsl] + eb[m, j, pl.ds(DH + k * 16, 16)],
                        0.0)

        pltpu.async_copy(mb, agg.at[dst_v.at[g]], sm, add=True)

    for g0 in range(NSLOT):
        start(g0, g0)

    def chunkn(t, carry):
        g = t * NSLOT
        for u in range(NSLOT):
            finish(g + u, u)

            @pl.when(g + u + NSLOT < NCHUNK)
            def _(u=u):
                start(g + u + NSLOT, u)

        return carry

    lax.fori_loop(0, NCHUNK // NSLOT, chunkn, 0)
    for r in range(NCHUNK - (NCHUNK // NSLOT) * NSLOT):
        finish((NCHUNK // NSLOT) * NSLOT + r, r)
    for k in range(NSLOT):
        pltpu.make_async_copy(mbufs[k], agg.at[dst_v.at[0]],
                              sem_ms[k]).wait()
    plsc.subcore_barrier()

    pltpu.sync_copy(agg.at[pl.ds(base_n, ROWS_PER_SUB), :],
                    out_hbm.at[c, pl.ds(base_n, ROWS_PER_SUB), :])


_sc_agg = pl.kernel(
    _sc_agg_body,
    out_type=jax.ShapeDtypeStruct((NC, NPAD, DH), jnp.float32),
    mesh=plsc.VectorSubcoreMesh(core_axis_name="c", subcore_axis_name="s"),
    scratch_types=(
        [pltpu.VMEM((NCHUNK, CHUNK), jnp.int32)] * 2
        + [pltpu.VMEM((CHUNK,), jnp.int32)] * NSLOT
        + [pltpu.VMEM((CHUNK, DH), jnp.float32)] * NSLOT
        + [pltpu.VMEM((4, CHUNK // 8, D), jnp.float32)] * NSLOT
        + [pltpu.VMEM((CHUNK, DH), jnp.float32)] * NSLOT
        + [pltpu.VMEM_SHARED((NPAD, DH), jnp.float32)]
        + [pltpu.SemaphoreType.DMA] * (3 * NSLOT)
    ),
    compiler_params=pltpu.CompilerParams(use_tc_tiling_on_sc=False),
)


def _node_mlp_body(out_relu, p_ref, h_ref, w1_ref, b1_ref, w2_ref, b2_ref,
                   y_ref):
    node = jnp.concatenate([p_ref[0], p_ref[1]], axis=-1) + h_ref[...]
    hid = jnp.maximum(
        jnp.dot(node, w1_ref[...], preferred_element_type=jnp.float32)
        + b1_ref[...], 0.0)
    y = (jnp.dot(hid, w2_ref[...], preferred_element_type=jnp.float32)
         + b2_ref[...])
    if out_relu:
        y = jnp.maximum(y, 0.0)
    y_ref[...] = y


_BN = 1000


def _node_mlp(parts, h, w1, b1, w2, b2, out_relu):
    return pl.pallas_call(
        functools.partial(_node_mlp_body, out_relu),
        grid=(N // _BN,),
        in_specs=[
            pl.BlockSpec((NC, _BN, DH), lambda i: (0, i, 0)),
            pl.BlockSpec((_BN, D), lambda i: (i, 0)),
            pl.BlockSpec((D, HID), lambda i: (0, 0)),
            pl.BlockSpec((1, HID), lambda i: (0, 0)),
            pl.BlockSpec((HID, D), lambda i: (0, 0)),
            pl.BlockSpec((1, D), lambda i: (0, 0)),
        ],
        out_specs=pl.BlockSpec((_BN, D), lambda i: (i, 0)),
        out_shape=jax.ShapeDtypeStruct((N, D), jnp.float32),
    )(parts, h, w1, b1.reshape(1, HID), w2, b2.reshape(1, D))


def kernel(x, edge_index, edge_attr, lw0, lb0, w10, b10, w20, b20,
           lw1, lb1, w11, b11, w21, b21):
    ea8 = edge_attr.reshape(E // 8, 8 * DE)
    e0 = _edge_lin(ea8, lw0, lb0)
    e1 = _edge_lin(ea8, lw1, lb1)
    src3 = edge_index[0].reshape(NS, NCHUNK, CHUNK)
    dst3 = edge_index[1].reshape(NS, NCHUNK, CHUNK)

    p0 = _sc_agg(x.reshape(NC * N, DH), *e0, src3, dst3)
    h1 = _node_mlp(p0, x, w10, b10, w20, b20, out_relu=True)

    p1 = _sc_agg(h1.reshape(NC * N, DH), *e1, src3, dst3)
    return _node_mlp(p1, h1, w11, b11, w21, b21, out_relu=False)


# speedup vs baseline: 1.1309x; 1.1309x over previous
import functools

import jax
import jax.numpy as jnp
from jax import lax
from jax.experimental import pallas as pl
from jax.experimental.pallas import tpu as pltpu
from jax.experimental.pallas import tpu_sc as plsc

N, E, D, DE, HID = 10000, 320000, 128, 16, 256

NC, NS = 2, 16
DH = D // NC
EPW = E // NS
CHUNK = 80
NCHUNK = EPW // CHUNK
NPAD = 10240
ROWS_PER_SUB = NPAD // NS


def _edge_lin_body(ea_ref, w_ref, b_ref, out0_ref, out1_ref, out2_ref,
                   out3_ref):
    ea = ea_ref[...]
    outs = (out0_ref, out1_ref, out2_ref, out3_ref)
    for c in range(NC):
        r = (jnp.dot(ea, w_ref[c], preferred_element_type=jnp.float32)
             + b_ref[c])
        for m in range(4):
            outs[m][c] = r[:, m * D:(m + 1) * D]


_BE8 = 1000


def _edge_lin(edge_attr8, lw, lb):
    eye8 = jnp.eye(8, dtype=jnp.float32)
    wp = jnp.stack([
        jnp.kron(eye8, lw[:, :DH]),
        jnp.kron(eye8, lw[:, DH:]),
    ])
    bp = jnp.stack([
        jnp.tile(lb[:DH], 8).reshape(1, 8 * DH),
        jnp.tile(lb[DH:], 8).reshape(1, 8 * DH),
    ])
    out = jax.ShapeDtypeStruct((NC, E // 8, D), jnp.float32)
    return pl.pallas_call(
        _edge_lin_body,
        grid=(E // 8 // _BE8,),
        in_specs=[
            pl.BlockSpec((_BE8, 8 * DE), lambda i: (i, 0)),
            pl.BlockSpec((NC, 8 * DE, 8 * DH), lambda i: (0, 0, 0)),
            pl.BlockSpec((NC, 1, 8 * DH), lambda i: (0, 0, 0)),
        ],
        out_specs=[pl.BlockSpec((NC, _BE8, D), lambda i: (0, i, 0))] * 4,
        out_shape=[out] * 4,
    )(edge_attr8, wp, bp)


NSLOT = 2


def _sc_agg_body(h_hbm, e0_hbm, e1_hbm, e2_hbm, e3_hbm, src_hbm, dst_hbm,
                 out_hbm,
                 src_v, dst_v,
                 idx0, idx1,
                 hbuf0, hbuf1,
                 ebuf0, ebuf1,
                 mbuf0, mbuf1,
                 agg,
                 sem_h0, sem_h1,
                 sem_e0, sem_e1,
                 sem_m0, sem_m1):
    e_hbms = (e0_hbm, e1_hbm, e2_hbm, e3_hbm)
    idxs = (idx0, idx1)
    hbufs = (hbuf0, hbuf1)
    ebufs = (ebuf0, ebuf1)
    mbufs = (mbuf0, mbuf1)
    sem_hs = (sem_h0, sem_h1)
    sem_es = (sem_e0, sem_e1)
    sem_ms = (sem_m0, sem_m1)
    c = lax.axis_index("c")
    s = lax.axis_index("s")

    pltpu.sync_copy(src_hbm.at[s], src_v)
    pltpu.sync_copy(dst_hbm.at[s], dst_v)

    zero16 = jnp.zeros((16,), jnp.float32)

    def zrow(i, carry):
        for j in range(DH // 16):
            mbuf0[i, pl.ds(j * 16, 16)] = zero16
        return carry

    lax.fori_loop(0, CHUNK, zrow, 0)
    base_n = s * ROWS_PER_SUB
    for r in range(ROWS_PER_SUB // CHUNK):
        pltpu.sync_copy(mbuf0, agg.at[pl.ds(base_n + r * CHUNK, CHUNK), :])
    plsc.subcore_barrier()

    edge_base = s * EPW
    bufs = tuple(
        (idxs[k], hbufs[k], ebufs[k], mbufs[k], sem_hs[k], sem_es[k],
         sem_ms[k])
        for k in range(NSLOT))

    def start(g, slot):
        idx, hb, eb, _, sh, se, _sm = bufs[slot]

        def irow(k, carry):
            dsl = pl.ds(k * 16, 16)
            idx[dsl] = src_v[g, dsl] * 2 + c
            return carry

        lax.fori_loop(0, CHUNK // 16, irow, 0)
        pltpu.async_copy(h_hbm.at[idx], hb, sh)
        off8 = (edge_base + g * CHUNK) // 8
        for m in range(4):
            pltpu.async_copy(
                e_hbms[m].at[c, pl.ds(off8, CHUNK // 8), :], eb.at[m], se)

    def finish(g, slot):
        idx, hb, eb, mb, sh, se, sm = bufs[slot]
        pltpu.make_async_copy(h_hbm.at[idx], hb, sh).wait()
        for m in range(4):
            pltpu.make_async_copy(
                e_hbms[m].at[c, pl.ds(0, CHUNK // 8), :], eb.at[m],
                se).wait()

        @pl.when(jnp.asarray(g >= NSLOT))
        def _():
            pltpu.make_async_copy(mb, agg.at[dst_v.at[g]], sm).wait()

        @plsc.parallel_loop(0, CHUNK // 8, unroll=2)
        def mrow(j):
            for m in range(4):
                l = 8 * j + 2 * m
                for k in range(DH // 16):
                    dsl = pl.ds(k * 16, 16)
                    mb[l, dsl] = jnp.maximum(
                        hb[l, dsl] + eb[m, j, dsl], 0.0)
                    mb[l + 1, dsl] = jnp.maximum(
                        hb[l + 1, dsl] + eb[m, j, pl.ds(DH + k * 16, 16)],
                        0.0)

        pltpu.async_copy(mb, agg.at[dst_v.at[g]], sm, add=True)

    for g0 in range(NSLOT):
        start(g0, g0)

    def chunkn(t, carry):
        g = t * NSLOT
        for u in range(NSLOT):
            finish(g + u, u)

            @pl.when(g + u + NSLOT < NCHUNK)
            def _(u=u):
                start(g + u + NSLOT, u)

        return carry

    lax.fori_loop(0, NCHUNK // NSLOT, chunkn, 0)
    for r in range(NCHUNK - (NCHUNK // NSLOT) * NSLOT):
        finish((NCHUNK // NSLOT) * NSLOT + r, r)
    for k in range(NSLOT):
        pltpu.make_async_copy(mbufs[k], agg.at[dst_v.at[0]],
                              sem_ms[k]).wait()
    plsc.subcore_barrier()

    pltpu.sync_copy(agg.at[pl.ds(base_n, ROWS_PER_SUB), :],
                    out_hbm.at[c, pl.ds(base_n, ROWS_PER_SUB), :])


_sc_agg = pl.kernel(
    _sc_agg_body,
    out_type=jax.ShapeDtypeStruct((NC, NPAD, DH), jnp.float32),
    mesh=plsc.VectorSubcoreMesh(core_axis_name="c", subcore_axis_name="s"),
    scratch_types=(
        [pltpu.VMEM((NCHUNK, CHUNK), jnp.int32)] * 2
        + [pltpu.VMEM((CHUNK,), jnp.int32)] * NSLOT
        + [pltpu.VMEM((CHUNK, DH), jnp.float32)] * NSLOT
        + [pltpu.VMEM((4, CHUNK // 8, D), jnp.float32)] * NSLOT
        + [pltpu.VMEM((CHUNK, DH), jnp.float32)] * NSLOT
        + [pltpu.VMEM_SHARED((NPAD, DH), jnp.float32)]
        + [pltpu.SemaphoreType.DMA] * (3 * NSLOT)
    ),
    compiler_params=pltpu.CompilerParams(use_tc_tiling_on_sc=False),
)


def _node_mlp_body(out_relu, p_ref, h_ref, w1_ref, b1_ref, w2_ref, b2_ref,
                   y_ref):
    node = jnp.concatenate([p_ref[0], p_ref[1]], axis=-1) + h_ref[...]
    hid = jnp.maximum(
        jnp.dot(node, w1_ref[...], preferred_element_type=jnp.float32)
        + b1_ref[...], 0.0)
    y = (jnp.dot(hid, w2_ref[...], preferred_element_type=jnp.float32)
         + b2_ref[...])
    if out_relu:
        y = jnp.maximum(y, 0.0)
    y_ref[...] = y


_BN = 1000


def _node_mlp(parts, h, w1, b1, w2, b2, out_relu):
    return pl.pallas_call(
        functools.partial(_node_mlp_body, out_relu),
        grid=(N // _BN,),
        in_specs=[
            pl.BlockSpec((NC, _BN, DH), lambda i: (0, i, 0)),
            pl.BlockSpec((_BN, D), lambda i: (i, 0)),
            pl.BlockSpec((D, HID), lambda i: (0, 0)),
            pl.BlockSpec((1, HID), lambda i: (0, 0)),
            pl.BlockSpec((HID, D), lambda i: (0, 0)),
            pl.BlockSpec((1, D), lambda i: (0, 0)),
        ],
        out_specs=pl.BlockSpec((_BN, D), lambda i: (i, 0)),
        out_shape=jax.ShapeDtypeStruct((N, D), jnp.float32),
    )(parts, h, w1, b1.reshape(1, HID), w2, b2.reshape(1, D))


def kernel(x, edge_index, edge_attr, lw0, lb0, w10, b10, w20, b20,
           lw1, lb1, w11, b11, w21, b21):
    ea8 = edge_attr.reshape(E // 8, 8 * DE)
    e0 = _edge_lin(ea8, lw0, lb0)
    e1 = _edge_lin(ea8, lw1, lb1)
    src3 = edge_index[0].reshape(NS, NCHUNK, CHUNK)
    dst3 = edge_index[1].reshape(NS, NCHUNK, CHUNK)

    p0 = _sc_agg(x.reshape(NC * N, DH), *e0, src3, dst3)
    h1 = _node_mlp(p0, x, w10, b10, w20, b20, out_relu=True)

    p1 = _sc_agg(h1.reshape(NC * N, DH), *e1, src3, dst3)
    return _node_mlp(p1, h1, w11, b11, w21, b21, out_relu=False)
